# Initial kernel scaffold; baseline (speedup 1.0000x reference)
import jax
import jax.numpy as jnp
from jax.experimental import pallas as pl


def kernel(x, pe):
    raise NotImplementedError("write your pallas kernel here")


# speedup vs baseline: 1.4737x; 1.4737x over previous
import functools

import jax
import jax.numpy as jnp
from jax import lax
from jax.experimental import pallas as pl
from jax.experimental.pallas import tpu as pltpu
from jax.experimental.pallas import tpu_sc as plsc

_D = 4096
_B = 8192
_NC = 2
_NS = 16
_NW = _NC * _NS
_BPW = _B // _NW
_C = 8
_NCHUNK = _BPW // _C

_mesh = plsc.VectorSubcoreMesh(
    core_axis_name="c", subcore_axis_name="s", num_cores=_NC, num_subcores=_NS
)


@functools.partial(
    pl.kernel,
    out_type=jax.ShapeDtypeStruct((_B, _D), jnp.float32),
    mesh=_mesh,
    scratch_types=[
        pltpu.VMEM((_BPW,), jnp.int32),
        pltpu.VMEM((_C, _D), jnp.float32),
        pltpu.SemaphoreType.DMA,
    ],
)
def _sc_gather(table_hbm, idx_hbm, out_hbm, idx_v, buf_v, sem):
    wid = lax.axis_index("s") * _NC + lax.axis_index("c")
    base = wid * _BPW
    pltpu.sync_copy(idx_hbm.at[pl.ds(base, _BPW)], idx_v)

    def chunk(g, carry):
        off = g * _C
        pltpu.async_copy(
            table_hbm.at[idx_v.at[pl.ds(off, _C)]], buf_v, sem
        ).wait()
        pltpu.sync_copy(buf_v, out_hbm.at[pl.ds(base + off, _C)])
        return carry

    lax.fori_loop(0, _NCHUNK, chunk, 0)


def kernel(x, pe):
    return _sc_gather(pe, x)


# speedup vs baseline: 1.6723x; 1.1348x over previous
import functools

import jax
import jax.numpy as jnp
from jax import lax
from jax.experimental import pallas as pl
from jax.experimental.pallas import tpu as pltpu
from jax.experimental.pallas import tpu_sc as plsc

_D = 4096
_B = 8192
_NC = 2
_NS = 16
_NW = _NC * _NS
_BPW = _B // _NW
_C = 8
_NCHUNK = _BPW // _C

_mesh = plsc.VectorSubcoreMesh(
    core_axis_name="c", subcore_axis_name="s", num_cores=_NC, num_subcores=_NS
)


_NBUF = 2


@functools.partial(
    pl.kernel,
    out_type=jax.ShapeDtypeStruct((_B, _D), jnp.float32),
    mesh=_mesh,
    scratch_types=[
        pltpu.VMEM((_BPW,), jnp.int32),
        pltpu.VMEM((_NBUF, _C, _D), jnp.float32),
        pltpu.SemaphoreType.DMA,
        pltpu.SemaphoreType.DMA,
        pltpu.SemaphoreType.DMA,
        pltpu.SemaphoreType.DMA,
    ],
)
def _sc_gather(table_hbm, idx_hbm, out_hbm, idx_v, buf_v, gs0, gs1, ws0, ws1):
    gsems = (gs0, gs1)
    wsems = (ws0, ws1)
    wid = lax.axis_index("s") * _NC + lax.axis_index("c")
    base = wid * _BPW
    pltpu.sync_copy(idx_hbm.at[pl.ds(base, _BPW)], idx_v)

    def gather(g, b):
        return pltpu.make_async_copy(
            table_hbm.at[idx_v.at[pl.ds(g * _C, _C)]], buf_v.at[b], gsems[b]
        )

    def write(g, b):
        return pltpu.make_async_copy(
            buf_v.at[b], out_hbm.at[pl.ds(base + g * _C, _C)], wsems[b]
        )

    for b in range(_NBUF):
        gather(b, b).start()

    def outer(i, carry):
        g0 = i * _NBUF
        for b in range(_NBUF):
            g = g0 + b
            gather(g, b).wait()
            write(g, b).start()
        for b in range(_NBUF):
            g = g0 + b
            write(g, b).wait()
            gather(g + _NBUF, b).start()
        return carry

    lax.fori_loop(0, _NCHUNK // _NBUF - 1, outer, 0)

    g0 = _NCHUNK - _NBUF
    for b in range(_NBUF):
        g = g0 + b
        gather(g, b).wait()
        write(g, b).start()
    for b in range(_NBUF):
        write(g0 + b, b).wait()


def kernel(x, pe):
    return _sc_gather(pe, x)
